# baseline (speedup 1.0000x reference)
import functools

import jax
import jax.numpy as jnp
from jax.experimental import pallas as pl
from jax.experimental.pallas import tpu as pltpu

N = 10000
D = 128
AVG_D_LOG = 3.5
EPS = 1e-5
ROWS = 1000


def _t1_body(sum_ref, sq_ref, mx_ref, mn_ref, deg_ref, wt_ref, b_ref,
             raw_ref, cs_ref, csq_ref, acc1, acc2):
    i = pl.program_id(0)
    deg = deg_ref[...]
    degc = jnp.maximum(deg, 1.0)
    mean = sum_ref[...] / degc
    meansq = sq_ref[...] / degc
    std = jnp.sqrt(jnp.maximum(meansq - mean * mean, 0.0) + EPS)
    has = deg > 0.0
    mx = jnp.where(has, mx_ref[...], 0.0)
    mn = jnp.where(has, mn_ref[...], 0.0)
    agg = jnp.concatenate([mean, mx, mn, std], axis=1)
    logd = jnp.log(degc + 1.0)
    hs = jnp.concatenate([agg, agg * (logd / AVG_D_LOG), agg * (AVG_D_LOG / logd)],
                         axis=1)
    raw = jnp.dot(hs, wt_ref[...], preferred_element_type=jnp.float32) + b_ref[...]
    raw_ref[...] = raw

    @pl.when(i == 0)
    def _init():
        acc1[...] = jnp.zeros_like(acc1)
        acc2[...] = jnp.zeros_like(acc2)

    acc1[...] += jnp.sum(raw, axis=0, keepdims=True)
    acc2[...] += jnp.sum(raw * raw, axis=0, keepdims=True)

    @pl.when(i == pl.num_programs(0) - 1)
    def _fin():
        cs_ref[...] = acc1[...]
        csq_ref[...] = acc2[...]


def _t2_body(raw_ref, h_ref, cs_ref, csq_ref, g_ref, bt_ref, out_ref):
    mu = cs_ref[...] / N
    var = csq_ref[...] / N - mu * mu
    inv = jax.lax.rsqrt(var + 1e-5)
    y = (raw_ref[...] - mu) * inv * g_ref[...] + bt_ref[...]
    out_ref[...] = jnp.maximum(y, 0.0) + h_ref[...]


def _posttrans(s, sq, mx, mn, deg, h, Wt, b, gamma, beta):
    grid = N // ROWS
    row = lambda i: (i, 0)
    fixed = lambda i: (0, 0)
    raw, cs, csq = pl.pallas_call(
        _t1_body,
        grid=(grid,),
        in_specs=[
            pl.BlockSpec((ROWS, D), row),
            pl.BlockSpec((ROWS, D), row),
            pl.BlockSpec((ROWS, D), row),
            pl.BlockSpec((ROWS, D), row),
            pl.BlockSpec((ROWS, 1), row),
            pl.BlockSpec((12 * D, D), fixed),
            pl.BlockSpec((1, D), fixed),
        ],
        out_specs=[
            pl.BlockSpec((ROWS, D), row),
            pl.BlockSpec((1, D), fixed),
            pl.BlockSpec((1, D), fixed),
        ],
        out_shape=[
            jax.ShapeDtypeStruct((N, D), jnp.float32),
            jax.ShapeDtypeStruct((1, D), jnp.float32),
            jax.ShapeDtypeStruct((1, D), jnp.float32),
        ],
        scratch_shapes=[
            pltpu.VMEM((1, D), jnp.float32),
            pltpu.VMEM((1, D), jnp.float32),
        ],
    )(s, sq, mx, mn, deg, Wt, b)
    out = pl.pallas_call(
        _t2_body,
        grid=(grid,),
        in_specs=[
            pl.BlockSpec((ROWS, D), row),
            pl.BlockSpec((ROWS, D), row),
            pl.BlockSpec((1, D), fixed),
            pl.BlockSpec((1, D), fixed),
            pl.BlockSpec((1, D), fixed),
            pl.BlockSpec((1, D), fixed),
        ],
        out_specs=pl.BlockSpec((ROWS, D), row),
        out_shape=jax.ShapeDtypeStruct((N, D), jnp.float32),
    )(raw, h, cs, csq, gamma, beta)
    return out


def _aggregate_jnp(h, edge_index):
    src = edge_index[0]
    dst = edge_index[1]
    m = h[src]
    ones = jnp.ones((m.shape[0],), dtype=h.dtype)
    deg = jax.ops.segment_sum(ones, dst, num_segments=N)
    s = jax.ops.segment_sum(m, dst, num_segments=N)
    sq = jax.ops.segment_sum(m * m, dst, num_segments=N)
    mx = jax.ops.segment_max(m, dst, num_segments=N)
    mn = jax.ops.segment_min(m, dst, num_segments=N)
    mx = jnp.where(deg[:, None] > 0, mx, 0.0)
    mn = jnp.where(deg[:, None] > 0, mn, 0.0)
    return s, sq, mx, mn, deg


def kernel(h, edge_index, W, b, gamma, beta):
    s, sq, mx, mn, deg = _aggregate_jnp(h, edge_index)
    Wt = W.T.reshape(12 * D, D)
    return _posttrans(s, sq, mx, mn, deg.reshape(N, 1), h,
                      Wt, b.reshape(1, D), gamma.reshape(1, D),
                      beta.reshape(1, D))
